# Initial kernel scaffold; baseline (speedup 1.0000x reference)
import jax
import jax.numpy as jnp
from jax.experimental import pallas as pl


def kernel(x, w_gate, weight, Wd, bd, Wu, bu):
    raise NotImplementedError("write your pallas kernel here")


# speedup vs baseline: 2.0531x; 2.0531x over previous
import jax
import jax.numpy as jnp
from jax.experimental import pallas as pl
from jax.experimental.pallas import tpu as pltpu

_INTERPRET = False


def _adapter_call(x2, Wd, bd2, Wu, bu2, w_gate, B, S, D, E, r):
    ST = min(512, S)
    nst = (B * S) // ST
    per_b = S // ST
    prec = jax.lax.Precision.HIGHEST

    def body(x_ref, wd_ref, bd_ref, wu_ref, bu_ref, wg_ref, xb_ref, lg_ref):
        i = pl.program_id(0)
        xr = x_ref[...]
        dn = jax.lax.dot_general(xr, wd_ref[...], (((1,), (0,)), ((), ())),
                                 preferred_element_type=jnp.float32,
                                 precision=prec)
        dn = jnp.maximum(dn + bd_ref[...], 0.0)
        up = jax.lax.dot_general(dn, wu_ref[...], (((1,), (0,)), ((), ())),
                                 preferred_element_type=jnp.float32,
                                 precision=prec)
        xn = xr + (up + bu_ref[...])
        xb_ref[...] = xn.astype(jnp.bfloat16)
        rs = jnp.sum(xn, axis=0, keepdims=True)
        lg = jax.lax.dot_general(rs, wg_ref[...], (((1,), (0,)), ((), ())),
                                 preferred_element_type=jnp.float32,
                                 precision=prec)
        lg = lg.reshape(1, 1, E)

        @pl.when(i % per_b == 0)
        def _():
            lg_ref[...] = lg

        @pl.when(i % per_b != 0)
        def _():
            lg_ref[...] += lg

    return pl.pallas_call(
        body,
        grid=(nst,),
        in_specs=[
            pl.BlockSpec((ST, D), lambda i: (i, 0)),
            pl.BlockSpec((D, r), lambda i: (0, 0)),
            pl.BlockSpec((1, r), lambda i: (0, 0)),
            pl.BlockSpec((r, D), lambda i: (0, 0)),
            pl.BlockSpec((1, D), lambda i: (0, 0)),
            pl.BlockSpec((D, E), lambda i: (0, 0)),
        ],
        out_specs=[
            pl.BlockSpec((ST, D), lambda i: (i, 0)),
            pl.BlockSpec((1, 1, E), lambda i: (i // per_b, 0, 0)),
        ],
        out_shape=[
            jax.ShapeDtypeStruct((B * S, D), jnp.bfloat16),
            jax.ShapeDtypeStruct((B, 1, E), jnp.float32),
        ],
        interpret=_INTERPRET,
    )(x2, Wd, bd2, Wu, bu2, w_gate)


def _gating_call(logits, B, S, E, K):
    def body(lg_ref, gates_ref, idx_ref, loss_ref):
        lg = lg_ref[...] * (1.0 / S)
        iota = jax.lax.broadcasted_iota(jnp.int32, (B, E), 1)
        work = lg
        vals, idxs = [], []
        for _ in range(K):
            m = jnp.max(work, axis=1, keepdims=True)
            im = jnp.min(jnp.where(work == m, iota, E), axis=1, keepdims=True)
            vals.append(m)
            idxs.append(im)
            work = jnp.where(iota == im, -1e30, work)
        v0, v1 = vals
        i0, i1 = idxs
        mmax = jnp.maximum(v0, v1)
        e0 = jnp.exp(v0 - mmax)
        e1 = jnp.exp(v1 - mmax)
        zs = e0 + e1
        g0 = e0 / zs
        g1 = e1 / zs
        gf = (jnp.where(iota == i0, g0, 0.0)
              + jnp.where(iota == i1, g1, 0.0))
        load = jnp.sum((gf > 0).astype(jnp.float32), axis=0)
        imp = jnp.sum(gf, axis=0)

        def cv(v):
            sv = jnp.sum(v)
            svv = jnp.sum(v * v)
            mean = sv / E
            var = (svv - E * mean * mean) / (E - 1)
            return var / (mean * mean + 1e-10)

        loss = (cv(imp) + cv(load)) * 0.01
        gates_ref[...] = jnp.concatenate([g0, g1], axis=1)
        idx_ref[...] = jnp.concatenate([i0, i1], axis=1)
        loss_ref[...] = loss.reshape(1, 1)

    return pl.pallas_call(
        body,
        in_specs=[pl.BlockSpec((B, E), lambda: (0, 0))],
        out_specs=[
            pl.BlockSpec((B, K), lambda: (0, 0)),
            pl.BlockSpec((B, K), lambda: (0, 0)),
            pl.BlockSpec((1, 1), lambda: (0, 0)),
        ],
        out_shape=[
            jax.ShapeDtypeStruct((B, K), jnp.float32),
            jax.ShapeDtypeStruct((B, K), jnp.int32),
            jax.ShapeDtypeStruct((1, 1), jnp.float32),
        ],
        interpret=_INTERPRET,
    )(logits)


def _bmm_call(idx, xb2, weight, gates3, B, S, D, E, K):
    OT = min(1024, D)
    JT = min(512, D)
    grid = (B, D // OT, D // JT)

    def body(idx_ref, x_ref, w0_ref, w1_ref, g_ref, y_ref):
        j = pl.program_id(2)
        g0 = g_ref[0, 0, 0]
        g1 = g_ref[0, 0, 1]
        ew = w0_ref[0] * g0 + w1_ref[0] * g1
        ewb = ew.astype(jnp.bfloat16)
        xb = x_ref[...]
        part = jax.lax.dot_general(xb, ewb, (((1,), (1,)), ((), ())),
                                   preferred_element_type=jnp.float32)

        @pl.when(j == 0)
        def _():
            y_ref[...] = part

        @pl.when(j > 0)
        def _():
            y_ref[...] += part

    grid_spec = pltpu.PrefetchScalarGridSpec(
        num_scalar_prefetch=1,
        grid=grid,
        in_specs=[
            pl.BlockSpec((S, JT), lambda b, o, j, idx_r: (b, j)),
            pl.BlockSpec((1, OT, JT), lambda b, o, j, idx_r: (idx_r[b, 0], o, j)),
            pl.BlockSpec((1, OT, JT), lambda b, o, j, idx_r: (idx_r[b, 1], o, j)),
            pl.BlockSpec((1, 1, K), lambda b, o, j, idx_r: (b, 0, 0)),
        ],
        out_specs=pl.BlockSpec((S, OT), lambda b, o, j, idx_r: (b, o)),
    )
    return pl.pallas_call(
        body,
        grid_spec=grid_spec,
        out_shape=jax.ShapeDtypeStruct((B * S, D), jnp.float32),
        interpret=_INTERPRET,
    )(idx, xb2, weight, weight, gates3)


def kernel(x, w_gate, weight, Wd, bd, Wu, bu):
    B, S, D = x.shape
    E = w_gate.shape[1]
    K = 2
    r = Wd.shape[1]

    x2 = x.reshape(B * S, D)
    bd2 = bd.reshape(1, r)
    bu2 = bu.reshape(1, D)

    xb2, logits3 = _adapter_call(x2, Wd, bd2, Wu, bu2, w_gate, B, S, D, E, r)
    gates, idx, loss = _gating_call(logits3.reshape(B, E), B, S, E, K)
    gates3 = gates.reshape(B, 1, K)
    y2 = _bmm_call(idx, xb2, weight, gates3, B, S, D, E, K)
    return (y2.reshape(B, S, D), loss.reshape(()))


# speedup vs baseline: 4.5221x; 2.2025x over previous
import jax
import jax.numpy as jnp
from jax.experimental import pallas as pl
from jax.experimental.pallas import tpu as pltpu

_INTERPRET = False


def _adapter_call(x2, Wd, bd2, Wu, bu2, w_gate, B, S, D, E, r):
    ST = min(512, S)
    nst = (B * S) // ST
    per_b = S // ST

    def body(x_ref, wd_ref, bd_ref, wu_ref, bu_ref, wg_ref, xb_ref, lg_ref):
        i = pl.program_id(0)
        xr = x_ref[...]
        wd = wd_ref[...]
        x_hi = xr.astype(jnp.bfloat16)
        x_lo = (xr - x_hi.astype(jnp.float32)).astype(jnp.bfloat16)
        wd_hi = wd.astype(jnp.bfloat16)
        wd_lo = (wd - wd_hi.astype(jnp.float32)).astype(jnp.bfloat16)
        cdims = (((1,), (0,)), ((), ()))

        def bdot(a, w):
            return jax.lax.dot_general(a, w, cdims,
                                       preferred_element_type=jnp.float32)

        dn = bdot(x_hi, wd_hi) + (bdot(x_lo, wd_hi) + bdot(x_hi, wd_lo))
        dn = jnp.maximum(dn + bd_ref[...], 0.0)
        up = jax.lax.dot_general(dn.astype(jnp.bfloat16),
                                 wu_ref[...].astype(jnp.bfloat16),
                                 (((1,), (0,)), ((), ())),
                                 preferred_element_type=jnp.float32)
        xn = xr + (up + bu_ref[...])
        xb_ref[...] = xn.astype(jnp.bfloat16)
        dns = jnp.sum(dn, axis=0, keepdims=True)
        rs_up = jax.lax.dot_general(dns, wu_ref[...], (((1,), (0,)), ((), ())),
                                    preferred_element_type=jnp.float32,
                                    precision=jax.lax.Precision.HIGHEST)
        rs = jnp.sum(xr, axis=0, keepdims=True) + rs_up + ST * bu_ref[...]
        lg = jax.lax.dot_general(rs, wg_ref[...], (((1,), (0,)), ((), ())),
                                 preferred_element_type=jnp.float32,
                                 precision=jax.lax.Precision.HIGHEST)
        lg = lg.reshape(1, 1, E)

        @pl.when(i % per_b == 0)
        def _():
            lg_ref[...] = lg

        @pl.when(i % per_b != 0)
        def _():
            lg_ref[...] += lg

    return pl.pallas_call(
        body,
        grid=(nst,),
        in_specs=[
            pl.BlockSpec((ST, D), lambda i: (i, 0)),
            pl.BlockSpec((D, r), lambda i: (0, 0)),
            pl.BlockSpec((1, r), lambda i: (0, 0)),
            pl.BlockSpec((r, D), lambda i: (0, 0)),
            pl.BlockSpec((1, D), lambda i: (0, 0)),
            pl.BlockSpec((D, E), lambda i: (0, 0)),
        ],
        out_specs=[
            pl.BlockSpec((ST, D), lambda i: (i, 0)),
            pl.BlockSpec((1, 1, E), lambda i: (i // per_b, 0, 0)),
        ],
        out_shape=[
            jax.ShapeDtypeStruct((B * S, D), jnp.bfloat16),
            jax.ShapeDtypeStruct((B, 1, E), jnp.float32),
        ],
        interpret=_INTERPRET,
    )(x2, Wd, bd2, Wu, bu2, w_gate)


def _gating_call(logits, B, S, E, K):
    def body(lg_ref, gates_ref, idx_ref, loss_ref):
        lg = lg_ref[...] * (1.0 / S)
        iota = jax.lax.broadcasted_iota(jnp.int32, (B, E), 1)
        work = lg
        vals, idxs = [], []
        for _ in range(K):
            m = jnp.max(work, axis=1, keepdims=True)
            im = jnp.min(jnp.where(work == m, iota, E), axis=1, keepdims=True)
            vals.append(m)
            idxs.append(im)
            work = jnp.where(iota == im, -1e30, work)
        v0, v1 = vals
        i0, i1 = idxs
        mmax = jnp.maximum(v0, v1)
        e0 = jnp.exp(v0 - mmax)
        e1 = jnp.exp(v1 - mmax)
        zs = e0 + e1
        g0 = e0 / zs
        g1 = e1 / zs
        gf = (jnp.where(iota == i0, g0, 0.0)
              + jnp.where(iota == i1, g1, 0.0))
        load = jnp.sum((gf > 0).astype(jnp.float32), axis=0)
        imp = jnp.sum(gf, axis=0)

        def cv(v):
            sv = jnp.sum(v)
            svv = jnp.sum(v * v)
            mean = sv / E
            var = (svv - E * mean * mean) / (E - 1)
            return var / (mean * mean + 1e-10)

        loss = (cv(imp) + cv(load)) * 0.01
        gates_ref[...] = jnp.concatenate([g0, g1], axis=1)
        idx_ref[...] = jnp.concatenate([i0, i1], axis=1)
        loss_ref[...] = loss.reshape(1, 1)

    return pl.pallas_call(
        body,
        in_specs=[pl.BlockSpec((B, E), lambda: (0, 0))],
        out_specs=[
            pl.BlockSpec((B, K), lambda: (0, 0)),
            pl.BlockSpec((B, K), lambda: (0, 0)),
            pl.BlockSpec((1, 1), lambda: (0, 0)),
        ],
        out_shape=[
            jax.ShapeDtypeStruct((B, K), jnp.float32),
            jax.ShapeDtypeStruct((B, K), jnp.int32),
            jax.ShapeDtypeStruct((1, 1), jnp.float32),
        ],
        interpret=_INTERPRET,
    )(logits)


def _bmm_call(idx, xb2, weight, gates3, B, S, D, E, K):
    OT = min(512, D)
    grid = (B, D // OT)

    def body(idx_ref, x_ref, w0_ref, w1_ref, g_ref, y_ref):
        g0 = g_ref[0, 0, 0]
        g1 = g_ref[0, 0, 1]
        ew = w0_ref[0] * g0 + w1_ref[0] * g1
        ewb = ew.astype(jnp.bfloat16)
        y_ref[...] = jax.lax.dot_general(x_ref[...], ewb,
                                         (((1,), (1,)), ((), ())),
                                         preferred_element_type=jnp.float32)

    grid_spec = pltpu.PrefetchScalarGridSpec(
        num_scalar_prefetch=1,
        grid=grid,
        in_specs=[
            pl.BlockSpec((S, D), lambda b, o, idx_r: (b, 0)),
            pl.BlockSpec((1, OT, D), lambda b, o, idx_r: (idx_r[b, 0], o, 0)),
            pl.BlockSpec((1, OT, D), lambda b, o, idx_r: (idx_r[b, 1], o, 0)),
            pl.BlockSpec((1, 1, K), lambda b, o, idx_r: (b, 0, 0)),
        ],
        out_specs=pl.BlockSpec((S, OT), lambda b, o, idx_r: (b, o)),
    )
    return pl.pallas_call(
        body,
        grid_spec=grid_spec,
        out_shape=jax.ShapeDtypeStruct((B * S, D), jnp.float32),
        interpret=_INTERPRET,
    )(idx, xb2, weight, weight, gates3)


def kernel(x, w_gate, weight, Wd, bd, Wu, bu):
    B, S, D = x.shape
    E = w_gate.shape[1]
    K = 2
    r = Wd.shape[1]

    x2 = x.reshape(B * S, D)
    bd2 = bd.reshape(1, r)
    bu2 = bu.reshape(1, D)

    xb2, logits3 = _adapter_call(x2, Wd, bd2, Wu, bu2, w_gate, B, S, D, E, r)
    gates, idx, loss = _gating_call(logits3.reshape(B, E), B, S, E, K)
    gates3 = gates.reshape(B, 1, K)
    y2 = _bmm_call(idx, xb2, weight, gates3, B, S, D, E, K)
    return (y2.reshape(B, S, D), loss.reshape(()))


# speedup vs baseline: 4.6327x; 1.0245x over previous
import jax
import jax.numpy as jnp
from jax.experimental import pallas as pl
from jax.experimental.pallas import tpu as pltpu

_INTERPRET = False


def _adapter_call(x, Wd, bd2, Wu, bu2, w_gate, B, S, D, E, r):
    ST = min(512, S)

    def body(x_ref, wd_ref, bd_ref, wu_ref, bu_ref, wg_ref, xb_ref, lg_ref):
        s = pl.program_id(1)
        xr = x_ref[0]
        wd = wd_ref[...]
        x_hi = xr.astype(jnp.bfloat16)
        x_lo = (xr - x_hi.astype(jnp.float32)).astype(jnp.bfloat16)
        wd_hi = wd.astype(jnp.bfloat16)
        wd_lo = (wd - wd_hi.astype(jnp.float32)).astype(jnp.bfloat16)
        cdims = (((1,), (0,)), ((), ()))

        def bdot(a, w):
            return jax.lax.dot_general(a, w, cdims,
                                       preferred_element_type=jnp.float32)

        dn = bdot(x_hi, wd_hi) + (bdot(x_lo, wd_hi) + bdot(x_hi, wd_lo))
        dn = jnp.maximum(dn + bd_ref[...], 0.0)
        up = jax.lax.dot_general(dn.astype(jnp.bfloat16),
                                 wu_ref[...].astype(jnp.bfloat16),
                                 (((1,), (0,)), ((), ())),
                                 preferred_element_type=jnp.float32)
        xn = xr + (up + bu_ref[...])
        xb_ref[0] = xn.astype(jnp.bfloat16)
        dns = jnp.sum(dn, axis=0, keepdims=True)
        rs_up = jax.lax.dot_general(dns, wu_ref[...], (((1,), (0,)), ((), ())),
                                    preferred_element_type=jnp.float32,
                                    precision=jax.lax.Precision.HIGHEST)
        rs = jnp.sum(xr, axis=0, keepdims=True) + rs_up + ST * bu_ref[...]
        lg = jax.lax.dot_general(rs, wg_ref[...], (((1,), (0,)), ((), ())),
                                 preferred_element_type=jnp.float32,
                                 precision=jax.lax.Precision.HIGHEST)
        lg = lg.reshape(1, 1, E)

        @pl.when(s == 0)
        def _():
            lg_ref[...] = lg

        @pl.when(s != 0)
        def _():
            lg_ref[...] += lg

    return pl.pallas_call(
        body,
        grid=(B, S // ST),
        in_specs=[
            pl.BlockSpec((1, ST, D), lambda b, s: (b, s, 0)),
            pl.BlockSpec((D, r), lambda b, s: (0, 0)),
            pl.BlockSpec((1, r), lambda b, s: (0, 0)),
            pl.BlockSpec((r, D), lambda b, s: (0, 0)),
            pl.BlockSpec((1, D), lambda b, s: (0, 0)),
            pl.BlockSpec((D, E), lambda b, s: (0, 0)),
        ],
        out_specs=[
            pl.BlockSpec((1, ST, D), lambda b, s: (b, s, 0)),
            pl.BlockSpec((1, 1, E), lambda b, s: (b, 0, 0)),
        ],
        out_shape=[
            jax.ShapeDtypeStruct((B, S, D), jnp.bfloat16),
            jax.ShapeDtypeStruct((B, 1, E), jnp.float32),
        ],
        interpret=_INTERPRET,
    )(x, Wd, bd2, Wu, bu2, w_gate)


def _gating_call(logits3, B, S, E, K):
    def body(lg_ref, gates_ref, idx_ref, loss_ref):
        lg = lg_ref[...].reshape(B, E) * (1.0 / S)
        iota = jax.lax.broadcasted_iota(jnp.int32, (B, E), 1)
        work = lg
        vals, idxs = [], []
        for _ in range(K):
            m = jnp.max(work, axis=1, keepdims=True)
            im = jnp.min(jnp.where(work == m, iota, E), axis=1, keepdims=True)
            vals.append(m)
            idxs.append(im)
            work = jnp.where(iota == im, -1e30, work)
        v0, v1 = vals
        i0, i1 = idxs
        mmax = jnp.maximum(v0, v1)
        e0 = jnp.exp(v0 - mmax)
        e1 = jnp.exp(v1 - mmax)
        zs = e0 + e1
        g0 = e0 / zs
        g1 = e1 / zs
        gf = (jnp.where(iota == i0, g0, 0.0)
              + jnp.where(iota == i1, g1, 0.0))
        load = jnp.sum((gf > 0).astype(jnp.float32), axis=0)
        imp = jnp.sum(gf, axis=0)

        def cv(v):
            sv = jnp.sum(v)
            svv = jnp.sum(v * v)
            mean = sv / E
            var = (svv - E * mean * mean) / (E - 1)
            return var / (mean * mean + 1e-10)

        loss = (cv(imp) + cv(load)) * 0.01
        gates_ref[...] = jnp.concatenate([g0, g1], axis=1).reshape(B, 1, K)
        idx_ref[...] = jnp.concatenate([i0, i1], axis=1)
        loss_ref[...] = loss.reshape(1, 1)

    return pl.pallas_call(
        body,
        in_specs=[pl.BlockSpec((B, 1, E), lambda: (0, 0, 0))],
        out_specs=[
            pl.BlockSpec((B, 1, K), lambda: (0, 0, 0)),
            pl.BlockSpec((B, K), lambda: (0, 0)),
            pl.BlockSpec((1, 1), lambda: (0, 0)),
        ],
        out_shape=[
            jax.ShapeDtypeStruct((B, 1, K), jnp.float32),
            jax.ShapeDtypeStruct((B, K), jnp.int32),
            jax.ShapeDtypeStruct((1, 1), jnp.float32),
        ],
        interpret=_INTERPRET,
    )(logits3)


def _bmm_call(idx, xb, weight, gates3, B, S, D, E, K):
    OT = min(512, D)
    grid = (B, D // OT)

    def body(idx_ref, x_ref, w0_ref, w1_ref, g_ref, y_ref):
        g0 = g_ref[0, 0, 0]
        g1 = g_ref[0, 0, 1]
        ew = w0_ref[0] * g0 + w1_ref[0] * g1
        ewb = ew.astype(jnp.bfloat16)
        MT = 512
        for m in range(S // MT):
            xs = x_ref[0, pl.ds(m * MT, MT), :]
            y_ref[0, pl.ds(m * MT, MT), :] = jax.lax.dot_general(
                xs, ewb, (((1,), (1,)), ((), ())),
                preferred_element_type=jnp.float32)

    grid_spec = pltpu.PrefetchScalarGridSpec(
        num_scalar_prefetch=1,
        grid=grid,
        in_specs=[
            pl.BlockSpec((1, S, D), lambda b, o, idx_r: (b, 0, 0)),
            pl.BlockSpec((1, OT, D), lambda b, o, idx_r: (idx_r[b, 0], o, 0)),
            pl.BlockSpec((1, OT, D), lambda b, o, idx_r: (idx_r[b, 1], o, 0)),
            pl.BlockSpec((1, 1, K), lambda b, o, idx_r: (b, 0, 0)),
        ],
        out_specs=pl.BlockSpec((1, S, OT), lambda b, o, idx_r: (b, 0, o)),
    )
    return pl.pallas_call(
        body,
        grid_spec=grid_spec,
        out_shape=jax.ShapeDtypeStruct((B, S, D), jnp.float32),
        interpret=_INTERPRET,
    )(idx, xb, weight, weight, gates3)


def kernel(x, w_gate, weight, Wd, bd, Wu, bu):
    B, S, D = x.shape
    E = w_gate.shape[1]
    K = 2
    r = Wd.shape[1]

    bd2 = bd.reshape(1, r)
    bu2 = bu.reshape(1, D)

    xb, logits3 = _adapter_call(x, Wd, bd2, Wu, bu2, w_gate, B, S, D, E, r)
    gates3, idx, loss = _gating_call(logits3, B, S, E, K)
    y = _bmm_call(idx, xb, weight, gates3, B, S, D, E, K)
    return (y, loss.reshape(()))


# speedup vs baseline: 4.8574x; 1.0485x over previous
import jax
import jax.numpy as jnp
from jax.experimental import pallas as pl
from jax.experimental.pallas import tpu as pltpu

_INTERPRET = False


def _adapter_call(x, Wd, bd2, Wu, bu2, B, S, D, r):
    ST = min(1024, S)
    nt = S // ST

    def body(x_ref, wd_ref, bd_ref, wu_ref, bu_ref, xb_ref, sx_ref, sd_ref):
        xr = x_ref[0]
        wd = wd_ref[...]
        x_hi = xr.astype(jnp.bfloat16)
        x_lo = (xr - x_hi.astype(jnp.float32)).astype(jnp.bfloat16)
        wd_hi = wd.astype(jnp.bfloat16)
        wd_lo = (wd - wd_hi.astype(jnp.float32)).astype(jnp.bfloat16)
        cdims = (((1,), (0,)), ((), ()))

        def bdot(a, w):
            return jax.lax.dot_general(a, w, cdims,
                                       preferred_element_type=jnp.float32)

        dn = bdot(x_hi, wd_hi) + (bdot(x_lo, wd_hi) + bdot(x_hi, wd_lo))
        dn = jnp.maximum(dn + bd_ref[...], 0.0)
        up = jax.lax.dot_general(dn.astype(jnp.bfloat16),
                                 wu_ref[...].astype(jnp.bfloat16),
                                 (((1,), (0,)), ((), ())),
                                 preferred_element_type=jnp.float32)
        xn = xr + (up + bu_ref[...])
        xb_ref[0] = xn.astype(jnp.bfloat16)
        sx_ref[0] = jnp.sum(xr, axis=0, keepdims=True)
        sd_ref[0] = jnp.sum(dn, axis=0, keepdims=True)

    return pl.pallas_call(
        body,
        grid=(B, nt),
        in_specs=[
            pl.BlockSpec((1, ST, D), lambda b, s: (b, s, 0)),
            pl.BlockSpec((D, r), lambda b, s: (0, 0)),
            pl.BlockSpec((1, r), lambda b, s: (0, 0)),
            pl.BlockSpec((r, D), lambda b, s: (0, 0)),
            pl.BlockSpec((1, D), lambda b, s: (0, 0)),
        ],
        out_specs=[
            pl.BlockSpec((1, ST, D), lambda b, s: (b, s, 0)),
            pl.BlockSpec((1, 1, D), lambda b, s: (b * nt + s, 0, 0)),
            pl.BlockSpec((1, 1, r), lambda b, s: (b * nt + s, 0, 0)),
        ],
        out_shape=[
            jax.ShapeDtypeStruct((B, S, D), jnp.bfloat16),
            jax.ShapeDtypeStruct((B * nt, 1, D), jnp.float32),
            jax.ShapeDtypeStruct((B * nt, 1, r), jnp.float32),
        ],
        interpret=_INTERPRET,
    )(x, Wd, bd2, Wu, bu2)


def _gating_call(sx, sd, Wu, bu2, w_gate, B, S, D, E, K, r, nt):
    NT = B * nt
    hp = jax.lax.Precision.HIGHEST

    def body(sx_ref, sd_ref, wu_ref, bu_ref, wg_ref,
             gates_ref, idx_ref, loss_ref):
        sxm = sx_ref[...].reshape(NT, D)
        sdm = sd_ref[...].reshape(NT, r)
        lg_x = jax.lax.dot_general(sxm, wg_ref[...], (((1,), (0,)), ((), ())),
                                   preferred_element_type=jnp.float32,
                                   precision=hp)
        wu_g = jax.lax.dot_general(wu_ref[...], wg_ref[...],
                                   (((1,), (0,)), ((), ())),
                                   preferred_element_type=jnp.float32,
                                   precision=hp)
        lg_d = jax.lax.dot_general(sdm, wu_g, (((1,), (0,)), ((), ())),
                                   preferred_element_type=jnp.float32,
                                   precision=hp)
        bu_g = jax.lax.dot_general(bu_ref[...], wg_ref[...],
                                   (((1,), (0,)), ((), ())),
                                   preferred_element_type=jnp.float32,
                                   precision=hp)
        lgt = lg_x + lg_d
        rows = jax.lax.broadcasted_iota(jnp.int32, (B, NT), 1)
        sel = (rows // nt ==
               jax.lax.broadcasted_iota(jnp.int32, (B, NT), 0))
        lg = jax.lax.dot_general(sel.astype(jnp.float32), lgt,
                                 (((1,), (0,)), ((), ())),
                                 preferred_element_type=jnp.float32,
                                 precision=hp)
        lg = (lg + S * bu_g) * (1.0 / S)
        iota = jax.lax.broadcasted_iota(jnp.int32, (B, E), 1)
        work = lg
        vals, idxs = [], []
        for _ in range(K):
            m = jnp.max(work, axis=1, keepdims=True)
            im = jnp.min(jnp.where(work == m, iota, E), axis=1, keepdims=True)
            vals.append(m)
            idxs.append(im)
            work = jnp.where(iota == im, -1e30, work)
        v0, v1 = vals
        i0, i1 = idxs
        mmax = jnp.maximum(v0, v1)
        e0 = jnp.exp(v0 - mmax)
        e1 = jnp.exp(v1 - mmax)
        zs = e0 + e1
        g0 = e0 / zs
        g1 = e1 / zs
        gf = (jnp.where(iota == i0, g0, 0.0)
              + jnp.where(iota == i1, g1, 0.0))
        load = jnp.sum((gf > 0).astype(jnp.float32), axis=0)
        imp = jnp.sum(gf, axis=0)

        def cv(v):
            sv = jnp.sum(v)
            svv = jnp.sum(v * v)
            mean = sv / E
            var = (svv - E * mean * mean) / (E - 1)
            return var / (mean * mean + 1e-10)

        loss = (cv(imp) + cv(load)) * 0.01
        gates_ref[...] = jnp.concatenate([g0, g1], axis=1).reshape(B, 1, K)
        idx_ref[...] = jnp.concatenate([i0, i1], axis=1)
        loss_ref[...] = loss.reshape(1, 1)

    return pl.pallas_call(
        body,
        in_specs=[
            pl.BlockSpec((NT, 1, D), lambda: (0, 0, 0)),
            pl.BlockSpec((NT, 1, r), lambda: (0, 0, 0)),
            pl.BlockSpec((r, D), lambda: (0, 0)),
            pl.BlockSpec((1, D), lambda: (0, 0)),
            pl.BlockSpec((D, E), lambda: (0, 0)),
        ],
        out_specs=[
            pl.BlockSpec((B, 1, K), lambda: (0, 0, 0)),
            pl.BlockSpec((B, K), lambda: (0, 0)),
            pl.BlockSpec((1, 1), lambda: (0, 0)),
        ],
        out_shape=[
            jax.ShapeDtypeStruct((B, 1, K), jnp.float32),
            jax.ShapeDtypeStruct((B, K), jnp.int32),
            jax.ShapeDtypeStruct((1, 1), jnp.float32),
        ],
        interpret=_INTERPRET,
    )(sx, sd, Wu, bu2, w_gate)


def _bmm_call(idx, xb, weight, gates3, B, S, D, E, K):
    OT = min(512, D)
    grid = (B, D // OT)

    def body(idx_ref, x_ref, w0_ref, w1_ref, g_ref, y_ref):
        g0 = g_ref[0, 0, 0]
        g1 = g_ref[0, 0, 1]
        ew = w0_ref[0] * g0 + w1_ref[0] * g1
        ewb = ew.astype(jnp.bfloat16)
        MT = min(512, S)
        for m in range(S // MT):
            xs = x_ref[0, pl.ds(m * MT, MT), :]
            y_ref[0, pl.ds(m * MT, MT), :] = jax.lax.dot_general(
                xs, ewb, (((1,), (1,)), ((), ())),
                preferred_element_type=jnp.float32)

    grid_spec = pltpu.PrefetchScalarGridSpec(
        num_scalar_prefetch=1,
        grid=grid,
        in_specs=[
            pl.BlockSpec((1, S, D), lambda b, o, idx_r: (b, 0, 0)),
            pl.BlockSpec((1, OT, D), lambda b, o, idx_r: (idx_r[b, 0], o, 0)),
            pl.BlockSpec((1, OT, D), lambda b, o, idx_r: (idx_r[b, 1], o, 0)),
            pl.BlockSpec((1, 1, K), lambda b, o, idx_r: (b, 0, 0)),
        ],
        out_specs=pl.BlockSpec((1, S, OT), lambda b, o, idx_r: (b, 0, o)),
    )
    return pl.pallas_call(
        body,
        grid_spec=grid_spec,
        out_shape=jax.ShapeDtypeStruct((B, S, D), jnp.float32),
        interpret=_INTERPRET,
    )(idx, xb, weight, weight, gates3)


def kernel(x, w_gate, weight, Wd, bd, Wu, bu):
    B, S, D = x.shape
    E = w_gate.shape[1]
    K = 2
    r = Wd.shape[1]
    nt = S // min(1024, S)

    bd2 = bd.reshape(1, r)
    bu2 = bu.reshape(1, D)

    xb, sx, sd = _adapter_call(x, Wd, bd2, Wu, bu2, B, S, D, r)
    gates3, idx, loss = _gating_call(sx, sd, Wu, bu2, w_gate,
                                     B, S, D, E, K, r, nt)
    y = _bmm_call(idx, xb, weight, gates3, B, S, D, E, K)
    return (y, loss.reshape(()))
